# baseline (speedup 1.0000x reference)
import jax
import jax.numpy as jnp
from jax import lax
from jax.experimental import pallas as pl
from jax.experimental.pallas import tpu as pltpu
from jax.experimental.pallas import tpu_sc as plsc

B = 16384
CTX = 20
D = 64
PD = 128
NC = 2
NS = 16
NW = NC * NS
CB = 256
TGT_CHUNKS = B // (NW * CB)
CTX_CHUNKS = (B * CTX) // (NW * CB)
BB = 256
NB = B // BB


def _sc_body(tgt_hbm, ctx_hbm, in_tab_hbm, out_tab_hbm, out_hbm,
             idx_v, rows_v, sem):
    wid = lax.axis_index("s") * NC + lax.axis_index("c")

    @pl.loop(0, TGT_CHUNKS)
    def _tgt(j):
        base = (wid * TGT_CHUNKS + j) * CB
        pltpu.sync_copy(tgt_hbm.at[pl.ds(base, CB)], idx_v)
        pltpu.async_copy(in_tab_hbm.at[idx_v], rows_v, sem).wait()
        pltpu.sync_copy(rows_v, out_hbm.at[pl.ds(base, CB)])

    @pl.loop(0, CTX_CHUNKS)
    def _ctx(j):
        base = (wid * CTX_CHUNKS + j) * CB
        pltpu.sync_copy(ctx_hbm.at[pl.ds(base, CB)], idx_v)
        pltpu.async_copy(out_tab_hbm.at[idx_v], rows_v, sem).wait()
        pltpu.sync_copy(rows_v, out_hbm.at[pl.ds(B + base, CB)])


def _tc_body(rows_ref, o_ref):
    i = pl.program_id(0)
    tgt = rows_ref[0].astype(jnp.float32)
    svs = [jnp.sum(tgt * rows_ref[1 + w].astype(jnp.float32),
                   axis=1, keepdims=True) for w in range(CTX)]
    m = svs[0]
    for s in svs[1:]:
        m = jnp.maximum(m, s)
    z = jnp.exp(svs[0] - m)
    for s in svs[1:]:
        z = z + jnp.exp(s - m)
    part = m + jnp.log(z) - svs[1]

    @pl.when(i == 0)
    def _():
        o_ref[0, 0] = 0.0

    o_ref[0, 0] += jnp.sum(part) * (1.0 / B)


@jax.jit
def kernel(target, context, in_embed, out_embed):
    in_tab = jnp.pad(in_embed.astype(jnp.bfloat16), ((0, 0), (0, PD - D)))
    out_tab = jnp.pad(out_embed.astype(jnp.bfloat16), ((0, 0), (0, PD - D)))
    tgt_idx = target.astype(jnp.int32)
    ctx_idx = context.astype(jnp.int32).T.reshape(-1)

    mesh = plsc.VectorSubcoreMesh(core_axis_name="c", subcore_axis_name="s")
    rows = pl.kernel(
        _sc_body,
        out_type=jax.ShapeDtypeStruct((B * (CTX + 1), PD), jnp.bfloat16),
        mesh=mesh,
        compiler_params=pltpu.CompilerParams(
            needs_layout_passes=False, use_tc_tiling_on_sc=False),
        scratch_types=[
            pltpu.VMEM((CB,), jnp.int32),
            pltpu.VMEM((CB, PD), jnp.bfloat16),
            pltpu.SemaphoreType.DMA,
        ],
    )(tgt_idx, ctx_idx, in_tab, out_tab)

    loss = pl.pallas_call(
        _tc_body,
        grid=(NB,),
        in_specs=[pl.BlockSpec((CTX + 1, BB, PD), lambda i: (0, i, 0))],
        out_specs=pl.BlockSpec((1, 1), lambda i: (0, 0),
                               memory_space=pltpu.SMEM),
        out_shape=jax.ShapeDtypeStruct((1, 1), jnp.float32),
    )(rows.reshape(CTX + 1, B, PD))
    return loss[0, 0]


# speedup vs baseline: 1.6588x; 1.6588x over previous
import jax
import jax.numpy as jnp
from jax import lax
from jax.experimental import pallas as pl
from jax.experimental.pallas import tpu as pltpu
from jax.experimental.pallas import tpu_sc as plsc

B = 16384
CTX = 20
D = 64
V = 1000000
NC = 2
NS = 16
NW = NC * NS
CB = 256
TGT_CHUNKS = B // (NW * CB)
CTX_CHUNKS = (B * CTX) // (NW * CB)
BB = 256
NB = B // BB
TBLK = 2048
NTB = (V + TBLK - 1) // TBLK


def _relayout_body(in_t_ref, out_t_ref, in_r_ref, out_r_ref):
    for src, dst in ((in_t_ref, in_r_ref), (out_t_ref, out_r_ref)):
        dst[...] = jnp.swapaxes(src[...], 0, 1)


def _sc_body(idx_hbm, in_tab_hbm, out_tab_hbm, out_hbm, idx_v, rows_v, sem):
    wid = lax.axis_index("s") * NC + lax.axis_index("c")

    @pl.loop(0, TGT_CHUNKS)
    def _tgt(j):
        base = (wid * TGT_CHUNKS + j) * CB
        pltpu.sync_copy(idx_hbm.at[pl.ds(base, CB)], idx_v)
        pltpu.async_copy(in_tab_hbm.at[idx_v], rows_v, sem).wait()
        pltpu.sync_copy(rows_v, out_hbm.at[pl.ds(base, CB)])

    @pl.loop(0, CTX_CHUNKS)
    def _ctx(j):
        base = (wid * CTX_CHUNKS + j) * CB
        pltpu.sync_copy(idx_hbm.at[pl.ds(B + base, CB)], idx_v)
        pltpu.async_copy(out_tab_hbm.at[idx_v], rows_v, sem).wait()
        pltpu.sync_copy(rows_v, out_hbm.at[pl.ds(B + base, CB)])


def _tc_body(rows_ref, o_ref):
    i = pl.program_id(0)

    tgt = rows_ref[0]
    svs = [jnp.sum(tgt * rows_ref[1 + w], axis=1, keepdims=True)
           for w in range(CTX)]
    m = svs[0]
    for s in svs[1:]:
        m = jnp.maximum(m, s)
    z = jnp.exp(svs[0] - m)
    for s in svs[1:]:
        z = z + jnp.exp(s - m)
    part = m + jnp.log(z) - svs[1]

    @pl.when(i == 0)
    def _():
        o_ref[0, 0] = 0.0

    o_ref[0, 0] += jnp.sum(part) * (1.0 / B)


@jax.jit
def kernel(target, context, in_embed, out_embed):
    tgt_idx = target.astype(jnp.int32)
    ctx_idx = context.astype(jnp.int32).T.reshape(-1)
    cat_idx = jnp.concatenate([tgt_idx, ctx_idx])

    in_rows, out_rows = pl.pallas_call(
        _relayout_body,
        grid=(NTB,),
        in_specs=[
            pl.BlockSpec((D, TBLK), lambda i: (0, i)),
            pl.BlockSpec((D, TBLK), lambda i: (0, i)),
        ],
        out_specs=[
            pl.BlockSpec((TBLK, D), lambda i: (i, 0)),
            pl.BlockSpec((TBLK, D), lambda i: (i, 0)),
        ],
        out_shape=[
            jax.ShapeDtypeStruct((V, D), jnp.float32),
            jax.ShapeDtypeStruct((V, D), jnp.float32),
        ],
    )(in_embed.T, out_embed.T)

    mesh = plsc.VectorSubcoreMesh(core_axis_name="c", subcore_axis_name="s")
    rows = pl.kernel(
        _sc_body,
        out_type=jax.ShapeDtypeStruct((B * (CTX + 1), D), jnp.float32),
        mesh=mesh,
        compiler_params=pltpu.CompilerParams(
            needs_layout_passes=False, use_tc_tiling_on_sc=False),
        scratch_types=[
            pltpu.VMEM((CB,), jnp.int32),
            pltpu.VMEM((CB, D), jnp.float32),
            pltpu.SemaphoreType.DMA,
        ],
    )(cat_idx, in_rows, out_rows)

    loss = pl.pallas_call(
        _tc_body,
        grid=(NB,),
        in_specs=[
            pl.BlockSpec((CTX + 1, BB, D), lambda i: (0, i, 0)),
        ],
        out_specs=pl.BlockSpec((1, 1), lambda i: (0, 0),
                               memory_space=pltpu.SMEM),
        out_shape=jax.ShapeDtypeStruct((1, 1), jnp.float32),
    )(rows.reshape(CTX + 1, B, D))
    return loss[0, 0]


# speedup vs baseline: 2.1380x; 1.2889x over previous
import jax
import jax.numpy as jnp
from jax import lax
from jax.experimental import pallas as pl
from jax.experimental.pallas import tpu as pltpu
from jax.experimental.pallas import tpu_sc as plsc

B = 16384
CTX = 20
D = 64
V = 1000000
PH = 500224
PD = 2 * D
NC = 2
NS = 16
NW = NC * NS
CB = 256
TGT_CHUNKS = B // (NW * CB)
CTX_CHUNKS = (B * CTX) // (NW * CB)
BB = 256
NB = B // BB
CBLK = 512
NTB = PH // CBLK


def _relayout_body(in_a, in_b, out_a, out_b, in_p_ref, out_p_ref):
    for lo, hi, dst in ((in_a, in_b, in_p_ref), (out_a, out_b, out_p_ref)):
        dst[...] = jnp.concatenate(
            [jnp.swapaxes(lo[...], 0, 1), jnp.swapaxes(hi[...], 0, 1)],
            axis=1)


def _sc_body(idx_hbm, in_tab_hbm, out_tab_hbm, out_hbm, idx_v, rows_v, sem):
    wid = lax.axis_index("s") * NC + lax.axis_index("c")

    @pl.loop(0, TGT_CHUNKS)
    def _tgt(j):
        base = (wid * TGT_CHUNKS + j) * CB
        pltpu.sync_copy(idx_hbm.at[pl.ds(base, CB)], idx_v)
        pltpu.async_copy(in_tab_hbm.at[idx_v], rows_v, sem).wait()
        pltpu.sync_copy(rows_v, out_hbm.at[pl.ds(base, CB)])

    @pl.loop(0, CTX_CHUNKS)
    def _ctx(j):
        base = (wid * CTX_CHUNKS + j) * CB
        pltpu.sync_copy(idx_hbm.at[pl.ds(B + base, CB)], idx_v)
        pltpu.async_copy(out_tab_hbm.at[idx_v], rows_v, sem).wait()
        pltpu.sync_copy(rows_v, out_hbm.at[pl.ds(B + base, CB)])


def _tc_body(rows_ref, half_ref, o_ref):
    i = pl.program_id(0)

    def half(s):
        x = rows_ref[s]
        h = half_ref[:, s:s + 1]
        return jnp.where(h == 1, x[:, D:PD], x[:, 0:D])

    tgt = half(0)
    svs = [jnp.sum(tgt * half(1 + w), axis=1, keepdims=True)
           for w in range(CTX)]
    m = svs[0]
    for s in svs[1:]:
        m = jnp.maximum(m, s)
    z = jnp.exp(svs[0] - m)
    for s in svs[1:]:
        z = z + jnp.exp(s - m)
    part = m + jnp.log(z) - svs[1]

    @pl.when(i == 0)
    def _():
        o_ref[0, 0] = 0.0

    o_ref[0, 0] += jnp.sum(part) * (1.0 / B)


@jax.jit
def kernel(target, context, in_embed, out_embed):
    tgt_idx = target.astype(jnp.int32)
    ctx_idx = context.astype(jnp.int32).T.reshape(-1)
    cat_idx = jnp.concatenate([tgt_idx, ctx_idx])
    phys_idx = jnp.where(cat_idx >= PH, cat_idx - PH, cat_idx)
    hlf = (cat_idx >= PH).astype(jnp.int32).reshape(CTX + 1, B).T

    in_pack, out_pack = pl.pallas_call(
        _relayout_body,
        grid=(NTB,),
        in_specs=[
            pl.BlockSpec((D, CBLK), lambda i: (0, i)),
            pl.BlockSpec((D, CBLK), lambda i: (0, i + NTB)),
            pl.BlockSpec((D, CBLK), lambda i: (0, i)),
            pl.BlockSpec((D, CBLK), lambda i: (0, i + NTB)),
        ],
        out_specs=[
            pl.BlockSpec((CBLK, PD), lambda i: (i, 0)),
            pl.BlockSpec((CBLK, PD), lambda i: (i, 0)),
        ],
        out_shape=[
            jax.ShapeDtypeStruct((PH, PD), jnp.float32),
            jax.ShapeDtypeStruct((PH, PD), jnp.float32),
        ],
    )(in_embed.T, in_embed.T, out_embed.T, out_embed.T)

    mesh = plsc.VectorSubcoreMesh(core_axis_name="c", subcore_axis_name="s")
    rows = pl.kernel(
        _sc_body,
        out_type=jax.ShapeDtypeStruct((B * (CTX + 1), PD), jnp.float32),
        mesh=mesh,
        compiler_params=pltpu.CompilerParams(
            needs_layout_passes=False, use_tc_tiling_on_sc=False),
        scratch_types=[
            pltpu.VMEM((CB,), jnp.int32),
            pltpu.VMEM((CB, PD), jnp.float32),
            pltpu.SemaphoreType.DMA,
        ],
    )(phys_idx, in_pack, out_pack)

    loss = pl.pallas_call(
        _tc_body,
        grid=(NB,),
        in_specs=[
            pl.BlockSpec((CTX + 1, BB, PD), lambda i: (0, i, 0)),
            pl.BlockSpec((BB, CTX + 1), lambda i: (i, 0)),
        ],
        out_specs=pl.BlockSpec((1, 1), lambda i: (0, 0),
                               memory_space=pltpu.SMEM),
        out_shape=jax.ShapeDtypeStruct((1, 1), jnp.float32),
    )(rows.reshape(CTX + 1, B, PD), hlf)
    return loss[0, 0]
